# baseline (speedup 1.0000x reference)
import functools

import jax
import jax.numpy as jnp
from jax import lax
from jax.experimental import pallas as pl
from jax.experimental.pallas import tpu as pltpu
from jax.experimental.pallas import tpu_sc as plsc

T = 2048
H = 768
FF = 1024
E = 64
CAP = 192
S = E * CAP
SPAD = S + 8
NC, NS = 2, 16
NW = NC * NS
BPW = T // NW
RCHUNK = 512


def _router_body(x_ref, wg_ref, slot_s_ref, slot_g_ref, gate_ref, cnt_ref):
    x = x_ref[...]
    logits = jnp.dot(x, wg_ref[...], preferred_element_type=jnp.float32)
    m = jnp.max(logits, axis=1, keepdims=True)
    ex = jnp.exp(logits - m)
    ssum = jnp.sum(ex, axis=1, keepdims=True)
    gate = 1.0 / ssum
    col = lax.broadcasted_iota(jnp.int32, (T, E), 1)
    idx = jnp.min(jnp.where(logits == m, col, E), axis=1, keepdims=True)
    oh = (col == idx).astype(jnp.float32)
    cnt_ref[...] = jnp.sum(oh, axis=0, keepdims=True).astype(jnp.int32)
    for b in range(T // RCHUNK):
        base = b * RCHUNK
        row_id = lax.broadcasted_iota(jnp.int32, (RCHUNK, T), 0) + base
        col_id = lax.broadcasted_iota(jnp.int32, (RCHUNK, T), 1)
        lb = (col_id < row_id).astype(jnp.bfloat16)
        cum = jnp.dot(lb, oh.astype(jnp.bfloat16),
                      preferred_element_type=jnp.float32)
        oh_b = oh[base:base + RCHUNK]
        pos = jnp.sum(cum * oh_b, axis=1, keepdims=True).astype(jnp.int32)
        idx_b = idx[base:base + RCHUNK]
        keep = pos < CAP
        slot = idx_b * CAP + pos
        slot_s_ref[base:base + RCHUNK] = jnp.where(keep, slot, S)
        slot_g_ref[base:base + RCHUNK] = jnp.where(keep, slot, S - 1)
        gate_ref[base:base + RCHUNK] = jnp.where(keep, gate[base:base + RCHUNK], 0.0)


_router = pl.pallas_call(
    _router_body,
    out_shape=(
        jax.ShapeDtypeStruct((T, 1), jnp.int32),
        jax.ShapeDtypeStruct((T, 1), jnp.int32),
        jax.ShapeDtypeStruct((T, 1), jnp.float32),
        jax.ShapeDtypeStruct((1, E), jnp.int32),
    ),
)


@functools.cache
def _sc_kernels():
    mesh = plsc.VectorSubcoreMesh(
        core_axis_name="c", subcore_axis_name="s", num_cores=NC, num_subcores=NS)
    scratch = [
        pltpu.VMEM((BPW,), jnp.int32),
        pltpu.VMEM((BPW, H), jnp.float32),
        pltpu.SemaphoreType.DMA,
    ]

    @functools.partial(
        pl.kernel,
        out_type=jax.ShapeDtypeStruct((SPAD, H), jnp.float32),
        mesh=mesh,
        scratch_types=scratch,
    )
    def dispatch(x_hbm, slot_hbm, xe_hbm, idx_v, rows_v, sem):
        wid = lax.axis_index("s") * NC + lax.axis_index("c")
        base = wid * BPW
        pltpu.sync_copy(slot_hbm.at[pl.ds(base, BPW)], idx_v)
        pltpu.sync_copy(x_hbm.at[pl.ds(base, BPW)], rows_v)
        pltpu.async_copy(rows_v, xe_hbm.at[idx_v], sem).wait()

    @functools.partial(
        pl.kernel,
        out_type=jax.ShapeDtypeStruct((T, H), jnp.float32),
        mesh=mesh,
        scratch_types=scratch,
    )
    def combine(ye_hbm, slot_hbm, y_hbm, idx_v, rows_v, sem):
        wid = lax.axis_index("s") * NC + lax.axis_index("c")
        base = wid * BPW
        pltpu.sync_copy(slot_hbm.at[pl.ds(base, BPW)], idx_v)
        pltpu.async_copy(ye_hbm.at[idx_v], rows_v, sem).wait()
        pltpu.sync_copy(rows_v, y_hbm.at[pl.ds(base, BPW)])

    return dispatch, combine


CT = 64
NT = CAP // CT


def _ffn_body(cnt_ref, xe_ref, w1_ref, b1_ref, w2_ref, b2_ref, ye_ref):
    e = pl.program_id(0)
    t = pl.program_id(1)

    @pl.when(t * CT < cnt_ref[e])
    def _():
        xb = xe_ref[...].astype(jnp.bfloat16)
        a = jnp.dot(xb, w1_ref[0].astype(jnp.bfloat16),
                    preferred_element_type=jnp.float32) + b1_ref[0]
        h = a * (1.0 / (1.0 + jnp.exp(-a)))
        ye_ref[...] = jnp.dot(h.astype(jnp.bfloat16), w2_ref[0].astype(jnp.bfloat16),
                              preferred_element_type=jnp.float32) + b2_ref[0]


def _cap_tile(e, t, cnt):
    last = jnp.maximum((cnt[e] + CT - 1) // CT - 1, 0)
    return e * NT + jnp.minimum(t, last)


_ffn = pl.pallas_call(
    _ffn_body,
    grid_spec=pltpu.PrefetchScalarGridSpec(
        num_scalar_prefetch=1,
        grid=(E, NT),
        in_specs=[
            pl.BlockSpec((CT, H), lambda e, t, cnt: (_cap_tile(e, t, cnt), 0)),
            pl.BlockSpec((1, H, FF), lambda e, t, cnt: (e, 0, 0)),
            pl.BlockSpec((1, 1, FF), lambda e, t, cnt: (e, 0, 0)),
            pl.BlockSpec((1, FF, H), lambda e, t, cnt: (e, 0, 0)),
            pl.BlockSpec((1, 1, H), lambda e, t, cnt: (e, 0, 0)),
        ],
        out_specs=pl.BlockSpec((CT, H), lambda e, t, cnt: (_cap_tile(e, t, cnt), 0)),
    ),
    out_shape=jax.ShapeDtypeStruct((S, H), jnp.float32),
)


def _scale_body(yr_ref, g_ref, out_ref):
    g = g_ref[...]
    out_ref[...] = jnp.where(g > 0.0, yr_ref[...] * g, 0.0)


_scale = pl.pallas_call(
    _scale_body,
    out_shape=jax.ShapeDtypeStruct((T, H), jnp.float32),
)


def kernel(hidden_states, Wg, W1, b1, W2, b2):
    orig_shape = hidden_states.shape
    x = hidden_states.reshape(T, H)
    dispatch, combine = _sc_kernels()
    slot_s, slot_g, gate, counts = _router(x, Wg)
    xe = dispatch(x, slot_s.reshape(T))
    ye = _ffn(counts.reshape(E), xe, W1, b1.reshape(E, 1, FF), W2, b2.reshape(E, 1, H))
    y_raw = combine(ye, slot_g.reshape(T))
    y = _scale(y_raw, gate)
    return y.reshape(orig_shape)


# speedup vs baseline: 1.6315x; 1.6315x over previous
import functools

import jax
import jax.numpy as jnp
from jax import lax
from jax.experimental import pallas as pl
from jax.experimental.pallas import tpu as pltpu
from jax.experimental.pallas import tpu_sc as plsc

T = 2048
H = 768
FF = 1024
E = 64
CAP = 192
S = E * CAP
SPAD = S + 8
NC, NS = 2, 16
NW = NC * NS
BPW = T // NW
RCHUNK = 512


def _router_body(x_ref, wg_ref, slot_s_ref, slot_g_ref, gate_ref):
    x = x_ref[...]
    logits = jnp.dot(x, wg_ref[...], preferred_element_type=jnp.float32)
    m = jnp.max(logits, axis=1, keepdims=True)
    ex = jnp.exp(logits - m)
    gate = 1.0 / jnp.sum(ex, axis=1, keepdims=True)
    col = lax.broadcasted_iota(jnp.int32, (T, E), 1)
    idx = jnp.min(jnp.where(logits == m, col, E), axis=1, keepdims=True)
    oh = (col == idx).astype(jnp.float32)
    cum = oh
    k = 1
    while k < T:
        cum = cum + jnp.concatenate(
            [jnp.zeros((k, E), jnp.float32), cum[:T - k]], axis=0)
        k *= 2
    cum = cum - oh
    pos = jnp.sum(cum * oh, axis=1, keepdims=True).astype(jnp.int32)
    keep = pos < CAP
    slot = idx * CAP + pos
    slot_s_ref[...] = jnp.where(keep, slot, S)
    slot_g_ref[...] = jnp.where(keep, slot, S - 1)
    gate_ref[...] = jnp.where(keep, gate, 0.0)


_router = pl.pallas_call(
    _router_body,
    out_shape=(
        jax.ShapeDtypeStruct((T, 1), jnp.int32),
        jax.ShapeDtypeStruct((T, 1), jnp.int32),
        jax.ShapeDtypeStruct((T, 1), jnp.float32),
    ),
)


@functools.cache
def _sc_kernels():
    mesh = plsc.VectorSubcoreMesh(
        core_axis_name="c", subcore_axis_name="s", num_cores=NC, num_subcores=NS)
    scratch = [
        pltpu.VMEM((BPW,), jnp.int32),
        pltpu.VMEM((BPW, H), jnp.float32),
        pltpu.SemaphoreType.DMA,
    ]

    @functools.partial(
        pl.kernel,
        out_type=jax.ShapeDtypeStruct((SPAD, H), jnp.float32),
        mesh=mesh,
        scratch_types=scratch,
    )
    def dispatch(x_hbm, slot_hbm, xe_hbm, idx_v, rows_v, sem):
        wid = lax.axis_index("s") * NC + lax.axis_index("c")
        base = wid * BPW
        pltpu.sync_copy(slot_hbm.at[pl.ds(base, BPW)], idx_v)
        pltpu.sync_copy(x_hbm.at[pl.ds(base, BPW)], rows_v)
        pltpu.async_copy(rows_v, xe_hbm.at[idx_v], sem).wait()

    @functools.partial(
        pl.kernel,
        out_type=jax.ShapeDtypeStruct((T, H), jnp.float32),
        mesh=mesh,
        scratch_types=scratch,
    )
    def combine(ye_hbm, slot_hbm, y_hbm, idx_v, rows_v, sem):
        wid = lax.axis_index("s") * NC + lax.axis_index("c")
        base = wid * BPW
        pltpu.sync_copy(slot_hbm.at[pl.ds(base, BPW)], idx_v)
        pltpu.async_copy(ye_hbm.at[idx_v], rows_v, sem).wait()
        pltpu.sync_copy(rows_v, y_hbm.at[pl.ds(base, BPW)])

    return dispatch, combine


CT = 64
NT = CAP // CT


def _ffn_body(xe_ref, w1_ref, b1_ref, w2_ref, b2_ref, ye_ref):
    xb = xe_ref[...].astype(jnp.bfloat16)
    a = jnp.dot(xb, w1_ref[0].astype(jnp.bfloat16),
                preferred_element_type=jnp.float32) + b1_ref[0]
    h = a * (1.0 / (1.0 + jnp.exp(-a)))
    ye_ref[...] = jnp.dot(h.astype(jnp.bfloat16), w2_ref[0].astype(jnp.bfloat16),
                          preferred_element_type=jnp.float32) + b2_ref[0]


_ffn = pl.pallas_call(
    _ffn_body,
    grid=(E,),
    in_specs=[
        pl.BlockSpec((CAP, H), lambda e: (e, 0)),
        pl.BlockSpec((1, H, FF), lambda e: (e, 0, 0)),
        pl.BlockSpec((1, 1, FF), lambda e: (e, 0, 0)),
        pl.BlockSpec((1, FF, H), lambda e: (e, 0, 0)),
        pl.BlockSpec((1, 1, H), lambda e: (e, 0, 0)),
    ],
    out_specs=pl.BlockSpec((CAP, H), lambda e: (e, 0)),
    out_shape=jax.ShapeDtypeStruct((S, H), jnp.float32),
)


def _scale_body(yr_ref, g_ref, out_ref):
    g = g_ref[...]
    out_ref[...] = jnp.where(g > 0.0, yr_ref[...] * g, 0.0)


_scale = pl.pallas_call(
    _scale_body,
    out_shape=jax.ShapeDtypeStruct((T, H), jnp.float32),
)


def kernel(hidden_states, Wg, W1, b1, W2, b2):
    orig_shape = hidden_states.shape
    x = hidden_states.reshape(T, H)
    dispatch, combine = _sc_kernels()
    slot_s, slot_g, gate = _router(x, Wg)
    xe = dispatch(x, slot_s.reshape(T))
    ye = _ffn(xe, W1, b1.reshape(E, 1, FF), W2, b2.reshape(E, 1, H))
    y_raw = combine(ye, slot_g.reshape(T))
    y = _scale(y_raw, gate)
    return y.reshape(orig_shape)
